# baseline (speedup 1.0000x reference)
import functools

import jax
import jax.numpy as jnp
from jax import lax
from jax.experimental import pallas as pl
from jax.experimental.pallas import tpu as pltpu
from jax.experimental.pallas import tpu_sc as plsc

N = 10000
D = 128
D_OUT = 2
E = 320000

NC = 2
NS = 16
NW = NC * NS

CHUNK = 64
NCHUNKS = E // CHUNK
K = 160
KQ = K // 4
ACC_ROWS = 10240
ROWS_PER_TILE_ZERO = ACC_ROWS // NS
ROWS_PER_TILE_OUT = 624
OUT_TAIL = N - NS * ROWS_PER_TILE_OUT


def _segsum_body(table_hbm, src_hbm, dst_hbm, out_hbm,
                 src_idx_v, dst_idx_v, r0, r1, r2, r3, acc_sh,
                 g0, g1, g2, g3, s0, s1, s2, s3):
    c = lax.axis_index("c")
    s = lax.axis_index("s")
    wid = c * NS + s
    R = (r0, r1, r2, r3)
    GS = (g0, g1, g2, g3)
    SS = (s0, s1, s2, s3)

    def _zrow(i, carry):
        for j in range(D // 16):
            r0[i, pl.ds(j * 16, 16)] = jnp.zeros((16,), jnp.float32)
        return carry
    lax.fori_loop(0, CHUNK, _zrow, 0)

    zbase = s * ROWS_PER_TILE_ZERO

    def _zcopy(r, carry):
        pltpu.sync_copy(r0, acc_sh.at[pl.ds(zbase + r * CHUNK, CHUNK)])
        return carry
    lax.fori_loop(0, ROWS_PER_TILE_ZERO // CHUNK, _zcopy, 0)

    plsc.subcore_barrier()

    def _gw(b):
        pltpu.make_async_copy(table_hbm.at[src_idx_v.at[0]], R[b], GS[b]).wait()

    def _sw(b):
        pltpu.make_async_copy(R[b], acc_sh.at[dst_idx_v.at[0]], SS[b]).wait()

    cb = wid * K

    def _quarter(q):
        pltpu.sync_copy(src_hbm.at[pl.ds(cb + q * KQ, KQ)], src_idx_v)
        pltpu.sync_copy(dst_hbm.at[pl.ds(cb + q * KQ, KQ)], dst_idx_v)
        for b in range(3):
            pltpu.async_copy(table_hbm.at[src_idx_v.at[b]], R[b], GS[b])

        def _grp(i, carry):
            for b in range(4):
                j = 4 * i + b
                t = (b + 3) % 4
                jn = jnp.minimum(j + 3, KQ - 1)
                _gw(b)
                if b == 0:
                    @pl.when(i > 0)
                    def _():
                        _sw(t)
                else:
                    _sw(t)
                pltpu.async_copy(R[b], acc_sh.at[dst_idx_v.at[j]], SS[b],
                                 add=True)
                pltpu.async_copy(table_hbm.at[src_idx_v.at[jn]], R[t], GS[t])
            return carry
        lax.fori_loop(0, KQ // 4, _grp, 0)
        for b in range(3):
            _gw(b)
        _sw(3)

    nq = jnp.where(wid == NW - 1, 1, 4)
    _quarter(0)
    for q in range(1, 4):
        @pl.when(q < nq)
        def _q():
            _quarter(q)

    plsc.subcore_barrier()

    pltpu.sync_copy(acc_sh.at[pl.ds(s * ROWS_PER_TILE_OUT, ROWS_PER_TILE_OUT)],
                    out_hbm.at[c, pl.ds(s * ROWS_PER_TILE_OUT, ROWS_PER_TILE_OUT)])

    @pl.when(s == 0)
    def _tail():
        pltpu.sync_copy(acc_sh.at[pl.ds(NS * ROWS_PER_TILE_OUT, OUT_TAIL)],
                        out_hbm.at[c, pl.ds(NS * ROWS_PER_TILE_OUT, OUT_TAIL)])


def _segment_sum_sc(table, src3, dst3):
    mesh = plsc.VectorSubcoreMesh(core_axis_name="c", subcore_axis_name="s")
    f = pl.kernel(
        _segsum_body,
        out_type=jax.ShapeDtypeStruct((NC, N, D), jnp.float32),
        mesh=mesh,
        scratch_types=[
            pltpu.VMEM((KQ, CHUNK), jnp.int32),
            pltpu.VMEM((KQ, CHUNK), jnp.int32),
            pltpu.VMEM((CHUNK, D), jnp.float32),
            pltpu.VMEM((CHUNK, D), jnp.float32),
            pltpu.VMEM((CHUNK, D), jnp.float32),
            pltpu.VMEM((CHUNK, D), jnp.float32),
            pltpu.VMEM_SHARED((ACC_ROWS, D), jnp.float32),
        ] + [pltpu.SemaphoreType.DMA] * 8,
    )
    return f(table, src3, dst3)


def _mlp1_body(p0_ref, p1_ref, x_ref, wa_ref, ba_ref, wb_ref, bb_ref, o_ref):
    hin = p0_ref[0] + p1_ref[0] + x_ref[...]
    t = jnp.dot(hin, wa_ref[...], preferred_element_type=jnp.float32) + ba_ref[...]
    t = jnp.maximum(t, 0.0)
    u = jnp.dot(t, wb_ref[...], preferred_element_type=jnp.float32) + bb_ref[...]
    o_ref[...] = jnp.maximum(u, 0.0)


def _mlp2_body(p0_ref, p1_ref, x_ref, wa_ref, ba_ref, wb_ref, bb_ref,
               wf_ref, bf_ref, o_ref):
    hin = p0_ref[0] + p1_ref[0] + x_ref[...]
    t = jnp.dot(hin, wa_ref[...], preferred_element_type=jnp.float32) + ba_ref[...]
    t = jnp.maximum(t, 0.0)
    h2 = jnp.dot(t, wb_ref[...], preferred_element_type=jnp.float32) + bb_ref[...]
    o_ref[...] = jnp.dot(h2, wf_ref[...], preferred_element_type=jnp.float32) + bf_ref[...]


_BR = 2000


def _row_spec():
    return pl.BlockSpec((_BR, D), lambda i: (i, 0))


def _part_spec(k):
    return pl.BlockSpec((1, _BR, D), lambda i, _k=k: (_k, i, 0))


def _w_spec():
    return pl.BlockSpec((D, D), lambda i: (0, 0))


def _b_spec():
    return pl.BlockSpec((1, D), lambda i: (0, 0))


def _mlp1(p, x, W1a, b1a, W1b, b1b):
    return pl.pallas_call(
        _mlp1_body,
        grid=(N // _BR,),
        in_specs=[_part_spec(0), _part_spec(1), _row_spec(),
                  _w_spec(), _b_spec(), _w_spec(), _b_spec()],
        out_specs=_row_spec(),
        out_shape=jax.ShapeDtypeStruct((N, D), jnp.float32),
    )(p, p, x, W1a, b1a.reshape(1, D), W1b, b1b.reshape(1, D))


def _mlp2(p, h, W2a, b2a, W2b, b2b, Wfc, bfc):
    return pl.pallas_call(
        _mlp2_body,
        grid=(N // _BR,),
        in_specs=[_part_spec(0), _part_spec(1), _row_spec(),
                  _w_spec(), _b_spec(), _w_spec(), _b_spec(),
                  pl.BlockSpec((D, D_OUT), lambda i: (0, 0)),
                  pl.BlockSpec((1, D_OUT), lambda i: (0, 0))],
        out_specs=pl.BlockSpec((_BR, D_OUT), lambda i: (i, 0)),
        out_shape=jax.ShapeDtypeStruct((N, D_OUT), jnp.float32),
    )(p, p, h, W2a, b2a.reshape(1, D), W2b, b2b.reshape(1, D),
      Wfc, bfc.reshape(1, D_OUT))


def kernel(x, edge_index, W1a, b1a, W1b, b1b, W2a, b2a, W2b, b2b, Wfc, bfc):
    src3 = edge_index[0].astype(jnp.int32).reshape(NCHUNKS, CHUNK)
    dst3 = edge_index[1].astype(jnp.int32).reshape(NCHUNKS, CHUNK)

    agg = _segment_sum_sc(x, src3, dst3)
    h = _mlp1(agg, x, W1a, b1a, W1b, b1b)
    agg2 = _segment_sum_sc(h, src3, dst3)
    return _mlp2(agg2, h, W2a, b2a, W2b, b2b, Wfc, bfc)


# speedup vs baseline: 1.0289x; 1.0289x over previous
import functools

import jax
import jax.numpy as jnp
from jax import lax
from jax.experimental import pallas as pl
from jax.experimental.pallas import tpu as pltpu
from jax.experimental.pallas import tpu_sc as plsc

N = 10000
D = 128
D_OUT = 2
E = 320000

NC = 2
NS = 16
NW = NC * NS

CHUNK = 64
NCHUNKS = E // CHUNK
K = 160
KQ = K // 4
NBUF = 4
ACC_ROWS = N
ROWS_PER_TILE_ZERO = ACC_ROWS // NS
ROWS_PER_TILE_OUT = 624
OUT_TAIL = N - NS * ROWS_PER_TILE_OUT


def _segsum_body(table_hbm, src_hbm, dst_hbm, out_hbm,
                 src_idx_v, dst_idx_v, r0, r1, r2, r3, acc_sh,
                 g0, g1, g2, g3, s0, s1, s2, s3):
    c = lax.axis_index("c")
    s = lax.axis_index("s")
    wid = c * NS + s
    R = (r0, r1, r2, r3)
    GS = (g0, g1, g2, g3)
    SS = (s0, s1, s2, s3)

    def _zrow(i, carry):
        for j in range(D // 16):
            r0[i, pl.ds(j * 16, 16)] = jnp.zeros((16,), jnp.float32)
        return carry
    lax.fori_loop(0, CHUNK, _zrow, 0)

    zbase = s * ROWS_PER_TILE_OUT
    zfull = ROWS_PER_TILE_OUT // CHUNK
    ztail = ROWS_PER_TILE_OUT - zfull * CHUNK

    def _zcopy(r, carry):
        pltpu.sync_copy(r0, acc_sh.at[pl.ds(zbase + r * CHUNK, CHUNK)])
        return carry
    lax.fori_loop(0, zfull, _zcopy, 0)
    pltpu.sync_copy(r0.at[pl.ds(0, ztail)],
                    acc_sh.at[pl.ds(zbase + zfull * CHUNK, ztail)])

    @pl.when(s == 0)
    def _ztail():
        pltpu.sync_copy(r0.at[pl.ds(0, OUT_TAIL)],
                        acc_sh.at[pl.ds(NS * ROWS_PER_TILE_OUT, OUT_TAIL)])

    plsc.subcore_barrier()

    def _sidx(j):
        return src_idx_v.at[j]

    def _gw(b):
        pltpu.make_async_copy(table_hbm.at[_sidx(0)], R[b], GS[b]).wait()

    def _sw(b):
        pltpu.make_async_copy(R[b], acc_sh.at[dst_idx_v.at[0]], SS[b]).wait()

    cb = wid * K

    def _quarter(q):
        pltpu.sync_copy(src_hbm.at[pl.ds(cb + q * KQ, KQ)], src_idx_v)
        pltpu.sync_copy(dst_hbm.at[pl.ds(cb + q * KQ, KQ)], dst_idx_v)
        for b in range(NBUF - 1):
            pltpu.async_copy(table_hbm.at[_sidx(b)], R[b], GS[b])

        def _grp(i, carry):
            for b in range(NBUF):
                j = NBUF * i + b
                t = (b + NBUF - 1) % NBUF
                jn = j + NBUF - 1
                _gw(b)
                if b == 0:
                    @pl.when(i > 0)
                    def _():
                        _sw(t)
                else:
                    _sw(t)
                pltpu.async_copy(R[b], acc_sh.at[dst_idx_v.at[j]], SS[b],
                                 add=True)

                @pl.when(jn < KQ)
                def _():
                    pltpu.async_copy(table_hbm.at[_sidx(jn)], R[t], GS[t])
            return carry
        lax.fori_loop(0, KQ // NBUF, _grp, 0)
        _sw(NBUF - 1)

    nq = jnp.where(wid == NW - 1, 1, 4)
    _quarter(0)
    for q in range(1, 4):
        @pl.when(q < nq)
        def _q():
            _quarter(q)

    plsc.subcore_barrier()

    pltpu.sync_copy(acc_sh.at[pl.ds(s * ROWS_PER_TILE_OUT, ROWS_PER_TILE_OUT)],
                    out_hbm.at[c, pl.ds(s * ROWS_PER_TILE_OUT, ROWS_PER_TILE_OUT)])

    @pl.when(s == 0)
    def _tail():
        pltpu.sync_copy(acc_sh.at[pl.ds(NS * ROWS_PER_TILE_OUT, OUT_TAIL)],
                        out_hbm.at[c, pl.ds(NS * ROWS_PER_TILE_OUT, OUT_TAIL)])


def _segment_sum_sc(table, src2, dst2):
    mesh = plsc.VectorSubcoreMesh(core_axis_name="c", subcore_axis_name="s")
    f = pl.kernel(
        _segsum_body,
        out_type=jax.ShapeDtypeStruct((NC, N, D), jnp.float32),
        mesh=mesh,
        scratch_types=[
            pltpu.VMEM((KQ, CHUNK), jnp.int32),
            pltpu.VMEM((KQ, CHUNK), jnp.int32),
        ] + [pltpu.VMEM((CHUNK, D), jnp.float32)] * NBUF + [
            pltpu.VMEM_SHARED((ACC_ROWS, D), jnp.float32),
        ] + [pltpu.SemaphoreType.DMA] * (2 * NBUF),
    )
    return f(table, src2, dst2)


def _mlp1_body(p0_ref, p1_ref, x_ref, wa_ref, ba_ref, wb_ref, bb_ref, o_ref):
    hin = p0_ref[0] + p1_ref[0] + x_ref[...]
    t = jnp.dot(hin, wa_ref[...], preferred_element_type=jnp.float32) + ba_ref[...]
    t = jnp.maximum(t, 0.0)
    u = jnp.dot(t, wb_ref[...], preferred_element_type=jnp.float32) + bb_ref[...]
    o_ref[...] = jnp.maximum(u, 0.0)


def _mlp2_body(p0_ref, p1_ref, x_ref, wa_ref, ba_ref, wb_ref, bb_ref,
               wf_ref, bf_ref, o_ref):
    hin = p0_ref[0] + p1_ref[0] + x_ref[...]
    t = jnp.dot(hin, wa_ref[...], preferred_element_type=jnp.float32) + ba_ref[...]
    t = jnp.maximum(t, 0.0)
    h2 = jnp.dot(t, wb_ref[...], preferred_element_type=jnp.float32) + bb_ref[...]
    o_ref[...] = jnp.dot(h2, wf_ref[...], preferred_element_type=jnp.float32) + bf_ref[...]


_BR = 2000


def _row_spec():
    return pl.BlockSpec((_BR, D), lambda i: (i, 0))


def _part_spec(k):
    return pl.BlockSpec((1, _BR, D), lambda i, _k=k: (_k, i, 0))


def _w_spec():
    return pl.BlockSpec((D, D), lambda i: (0, 0))


def _b_spec():
    return pl.BlockSpec((1, D), lambda i: (0, 0))


def _mlp1(p, x, W1a, b1a, W1b, b1b):
    return pl.pallas_call(
        _mlp1_body,
        grid=(N // _BR,),
        in_specs=[_part_spec(0), _part_spec(1), _row_spec(),
                  _w_spec(), _b_spec(), _w_spec(), _b_spec()],
        out_specs=_row_spec(),
        out_shape=jax.ShapeDtypeStruct((N, D), jnp.float32),
    )(p, p, x, W1a, b1a.reshape(1, D), W1b, b1b.reshape(1, D))


def _mlp2(p, h, W2a, b2a, W2b, b2b, Wfc, bfc):
    return pl.pallas_call(
        _mlp2_body,
        grid=(N // _BR,),
        in_specs=[_part_spec(0), _part_spec(1), _row_spec(),
                  _w_spec(), _b_spec(), _w_spec(), _b_spec(),
                  pl.BlockSpec((D, D_OUT), lambda i: (0, 0)),
                  pl.BlockSpec((1, D_OUT), lambda i: (0, 0))],
        out_specs=pl.BlockSpec((_BR, D_OUT), lambda i: (i, 0)),
        out_shape=jax.ShapeDtypeStruct((N, D_OUT), jnp.float32),
    )(p, p, h, W2a, b2a.reshape(1, D), W2b, b2b.reshape(1, D),
      Wfc, bfc.reshape(1, D_OUT))


def kernel(x, edge_index, W1a, b1a, W1b, b1b, W2a, b2a, W2b, b2b, Wfc, bfc):
    src3 = edge_index[0].astype(jnp.int32).reshape(NCHUNKS, CHUNK)
    dst3 = edge_index[1].astype(jnp.int32).reshape(NCHUNKS, CHUNK)

    agg = _segment_sum_sc(x, src3, dst3)
    h = _mlp1(agg, x, W1a, b1a, W1b, b1b)
    agg2 = _segment_sum_sc(h, src3, dst3)
    return _mlp2(agg2, h, W2a, b2a, W2b, b2b, Wfc, bfc)
